# baseline (speedup 1.0000x reference)
import jax
import jax.numpy as jnp
from jax import lax
from jax.experimental import pallas as pl
from jax.experimental.pallas import tpu as pltpu
from jax.experimental.pallas import tpu_sc as plsc

K = 16
D = 128

_ROW = 4096
_NROW = K * D * D * D // _ROW
_SEL = D * D * D // _ROW
_L = 16
_NC = 2


def _sc_body(dens_ref, conf_ref, out_ref, buf, conf_v, sem):
    wid = lax.axis_index("s") * _NC + lax.axis_index("c")
    base = wid * _L
    pltpu.sync_copy(conf_ref, conf_v)
    idx = conf_v[...] * _SEL + base + lax.broadcasted_iota(jnp.int32, (_L,), 0)
    pltpu.async_copy(dens_ref.at[idx], buf, sem).wait()
    pltpu.sync_copy(buf, out_ref.at[pl.ds(base, _L)])


def kernel(density, conformation):
    dens2d = density.reshape(_NROW, _ROW)
    conf_vec = jnp.full((_L,), conformation, jnp.int32)
    mesh = plsc.VectorSubcoreMesh(core_axis_name="c", subcore_axis_name="s")
    sc_call = pl.kernel(
        _sc_body,
        out_type=jax.ShapeDtypeStruct((_SEL, _ROW), jnp.float32),
        mesh=mesh,
        scratch_types=[
            pltpu.VMEM((_L, _ROW), jnp.float32),
            pltpu.VMEM((_L,), jnp.int32),
            pltpu.SemaphoreType.DMA,
        ],
    )
    out2d = sc_call(dens2d, conf_vec)
    return out2d.reshape(D, D, D)


# speedup vs baseline: 1.0026x; 1.0026x over previous
import jax
import jax.numpy as jnp
from jax import lax
from jax.experimental import pallas as pl
from jax.experimental.pallas import tpu as pltpu
from jax.experimental.pallas import tpu_sc as plsc

K = 16
D = 128

_ROW = 4096
_NROW = K * D * D * D // _ROW
_SEL = D * D * D // _ROW
_L = 16
_NC = 2


def _sc_body(dens_ref, conf_ref, out_ref, buf, conf_v, sem):
    wid = lax.axis_index("s") * _NC + lax.axis_index("c")
    base = wid * _L
    pltpu.sync_copy(conf_ref, conf_v)
    conf = conf_v[...][0]
    src = dens_ref.at[pl.ds(conf * _SEL + base, _L)]
    pltpu.async_copy(src, buf, sem).wait()
    pltpu.sync_copy(buf, out_ref.at[pl.ds(base, _L)])


def kernel(density, conformation):
    dens2d = density.reshape(_NROW, _ROW)
    conf_vec = jnp.full((_L,), conformation, jnp.int32)
    mesh = plsc.VectorSubcoreMesh(core_axis_name="c", subcore_axis_name="s")
    sc_call = pl.kernel(
        _sc_body,
        out_type=jax.ShapeDtypeStruct((_SEL, _ROW), jnp.float32),
        mesh=mesh,
        scratch_types=[
            pltpu.VMEM((_L, _ROW), jnp.float32),
            pltpu.VMEM((_L,), jnp.int32),
            pltpu.SemaphoreType.DMA,
        ],
    )
    out2d = sc_call(dens2d, conf_vec)
    return out2d.reshape(D, D, D)


# speedup vs baseline: 6.4060x; 6.3892x over previous
import jax
import jax.numpy as jnp
from jax import lax
from jax.experimental import pallas as pl
from jax.experimental.pallas import tpu as pltpu
from jax.experimental.pallas import tpu_sc as plsc

K = 16
D = 128

_L = 16
_NC = 2
_NW = 32
_RPW = D // _NW


def _sc_body(dens_ref, conf_ref, out_ref, buf, conf_v, sem):
    wid = lax.axis_index("s") * _NC + lax.axis_index("c")
    base = wid * _RPW
    pltpu.sync_copy(conf_ref, conf_v)
    conf = conf_v[...][0]
    src = dens_ref.at[conf, pl.ds(base, _RPW)]
    pltpu.async_copy(src, buf, sem).wait()
    pltpu.sync_copy(buf, out_ref.at[pl.ds(base, _RPW)])


def kernel(density, conformation):
    conf_vec = jnp.full((_L,), conformation, jnp.int32)
    mesh = plsc.VectorSubcoreMesh(core_axis_name="c", subcore_axis_name="s")
    sc_call = pl.kernel(
        _sc_body,
        out_type=jax.ShapeDtypeStruct((D, D, D), jnp.float32),
        mesh=mesh,
        scratch_types=[
            pltpu.VMEM((_RPW, D, D), jnp.float32),
            pltpu.VMEM((_L,), jnp.int32),
            pltpu.SemaphoreType.DMA,
        ],
    )
    return sc_call(density, conf_vec)
